# baseline (speedup 1.0000x reference)
import functools

import jax
import jax.numpy as jnp
from jax import lax
from jax.experimental import pallas as pl
from jax.experimental.pallas import tpu as pltpu
from jax.experimental.pallas import tpu_sc as plsc

VOCAB = 1000000
DIM = 64
B = 16384
F = 27
OUT_F = 1 + DIM + (F - 2)

NUM_CORES = 2
NUM_SUBCORES = 16
NW = NUM_CORES * NUM_SUBCORES
BPW = B // NW
LANES = 16
CHUNK = 128
NCHUNK = BPW // CHUNK
RPC = BPW // LANES


def _emb_body(x_hbm, w_hbm, out_hbm, x_v, idx_v, emb_v, out_v, sem):
    wid = lax.axis_index("s") * NUM_CORES + lax.axis_index("c")
    base = wid * BPW
    lanes = lax.iota(jnp.int32, LANES)

    pltpu.sync_copy(x_hbm.at[pl.ds(base, BPW)], x_v)

    col1 = jnp.full((LANES,), 1, jnp.int32)
    for i in range(RPC):
        r = lanes + (i * LANES)
        vals = plsc.load_gather(x_v, [r, col1])
        j, off = divmod(i * LANES, CHUNK)
        idx_v[j, pl.ds(off, LANES)] = vals.astype(jnp.int32)

    copies = [
        pltpu.async_copy(w_hbm.at[idx_v.at[j]],
                         emb_v.at[pl.ds(j * CHUNK, CHUNK)], sem)
        for j in range(NCHUNK)
    ]

    def pass_body(i, carry):
        r = lanes + i * LANES
        for c in range(F):
            if c == 1:
                continue
            dst = 0 if c == 0 else DIM - 1 + c
            vals = plsc.load_gather(x_v, [r, jnp.full((LANES,), c, jnp.int32)])
            plsc.store_scatter(
                out_v, [r, jnp.full((LANES,), dst, jnp.int32)], vals)
        return carry

    lax.fori_loop(0, RPC, pass_body, 0)

    for cp in copies:
        cp.wait()

    def emb_body(i, carry):
        for rr in range(LANES):
            row = jnp.full((LANES,), i * LANES + rr, jnp.int32)
            for k in range(DIM // LANES):
                cols = lanes + (k * LANES)
                vals = plsc.load_gather(emb_v, [row, cols])
                plsc.store_scatter(out_v, [row, cols + 1], vals)
        return carry

    lax.fori_loop(0, RPC, emb_body, 0)

    pltpu.sync_copy(out_v, out_hbm.at[pl.ds(base, BPW)])


@jax.jit
def kernel(x, W):
    mesh = plsc.VectorSubcoreMesh(core_axis_name="c", subcore_axis_name="s")
    fused = functools.partial(
        pl.kernel,
        mesh=mesh,
        compiler_params=pltpu.CompilerParams(
            needs_layout_passes=False, use_tc_tiling_on_sc=False),
        out_type=jax.ShapeDtypeStruct((B, OUT_F), jnp.float32),
        scratch_types=[
            pltpu.VMEM((BPW, F), jnp.float32),
            pltpu.VMEM((NCHUNK, CHUNK), jnp.int32),
            pltpu.VMEM((BPW, DIM), jnp.float32),
            pltpu.VMEM((BPW, OUT_F), jnp.float32),
            pltpu.SemaphoreType.DMA,
        ],
    )(_emb_body)
    return fused(x, W)


# speedup vs baseline: 1.0018x; 1.0018x over previous
import functools

import jax
import jax.numpy as jnp
from jax import lax
from jax.experimental import pallas as pl
from jax.experimental.pallas import tpu as pltpu
from jax.experimental.pallas import tpu_sc as plsc

VOCAB = 1000000
DIM = 64
B = 16384
F = 27
OUT_F = 1 + DIM + (F - 2)

NUM_CORES = 2
NUM_SUBCORES = 16
NW = NUM_CORES * NUM_SUBCORES
BPW = B // NW
LANES = 16
CHUNK = 128
NCHUNK = BPW // CHUNK
RPC = BPW // LANES


def _emb_body(x_hbm, w_hbm, out_hbm, x_v, idx_v, emb_v, out_v, sem):
    wid = lax.axis_index("s") * NUM_CORES + lax.axis_index("c")
    base = wid * BPW
    lanes = lax.iota(jnp.int32, LANES)

    pltpu.sync_copy(x_hbm.at[pl.ds(base, BPW)], x_v)

    col1 = jnp.full((LANES,), 1, jnp.int32)
    for i in range(RPC):
        r = lanes + (i * LANES)
        vals = plsc.load_gather(x_v, [r, col1])
        j, off = divmod(i * LANES, CHUNK)
        idx_v[j, pl.ds(off, LANES)] = vals.astype(jnp.int32)

    copies = [
        pltpu.async_copy(w_hbm.at[idx_v.at[j]],
                         emb_v.at[pl.ds(j * CHUNK, CHUNK)], sem)
        for j in range(NCHUNK)
    ]

    def pass_body(i, carry):
        r = lanes + i * LANES
        for c in range(F):
            if c == 1:
                continue
            dst = 0 if c == 0 else DIM - 1 + c
            vals = plsc.load_gather(x_v, [r, jnp.full((LANES,), c, jnp.int32)])
            plsc.store_scatter(
                out_v, [r, jnp.full((LANES,), dst, jnp.int32)], vals)
        return carry

    lax.fori_loop(0, RPC, pass_body, 0)

    for cp in copies:
        cp.wait()

    def emb_body(i, carry):
        for rr in range(LANES):
            row = jnp.full((LANES,), i * LANES + rr, jnp.int32)
            for k in range(DIM // LANES):
                cols = lanes + (k * LANES)
                vals = plsc.load_gather(emb_v, [row, cols])
                plsc.store_scatter(out_v, [row, cols + 1], vals)
        return carry

    lax.fori_loop(0, RPC, emb_body, 0)

    pltpu.sync_copy(out_v, out_hbm.at[pl.ds(base, BPW)])


@jax.jit
def kernel(x, W):
    mesh = plsc.VectorSubcoreMesh(core_axis_name="c", subcore_axis_name="s")
    fused = functools.partial(
        pl.kernel,
        mesh=mesh,
        compiler_params=pltpu.CompilerParams(
            needs_layout_passes=False, use_tc_tiling_on_sc=False,
            skip_device_barrier=True),
        out_type=jax.ShapeDtypeStruct((B, OUT_F), jnp.float32),
        scratch_types=[
            pltpu.VMEM((BPW, F), jnp.float32),
            pltpu.VMEM((NCHUNK, CHUNK), jnp.int32),
            pltpu.VMEM((BPW, DIM), jnp.float32),
            pltpu.VMEM((BPW, OUT_F), jnp.float32),
            pltpu.SemaphoreType.DMA,
        ],
    )(_emb_body)
    return fused(x, W)


# speedup vs baseline: 2.2180x; 2.2140x over previous
import functools

import jax
import jax.numpy as jnp
from jax import lax
from jax.experimental import pallas as pl
from jax.experimental.pallas import tpu as pltpu
from jax.experimental.pallas import tpu_sc as plsc

VOCAB = 1000000
DIM = 64
B = 16384
F = 27
OUT_F = 1 + DIM + (F - 2)
TB = 8
NBLK = VOCAB // TB

NUM_CORES = 2
NUM_SUBCORES = 16
NW = NUM_CORES * NUM_SUBCORES
BPW = B // NW
HALF = BPW // 2
LANES = 16
NCH = HALF // LANES


def _emb_body(x_hbm, w_hbm, out_hbm, x_v, ids_v, blk0, blk1, out_v,
              sem0, sem1):
    wid = lax.axis_index("s") * NUM_CORES + lax.axis_index("c")
    lanes = lax.iota(jnp.int32, LANES)
    threes = jnp.full((LANES,), 3, jnp.int32)
    sevens = jnp.full((LANES,), TB - 1, jnp.int32)
    zeros = jnp.full((LANES,), 0, jnp.int32)
    col1 = jnp.full((LANES,), 1, jnp.int32)

    def fire(c, buf, sem):
        ids16 = ids_v[pl.ds(c * LANES, LANES)]
        b16 = lax.shift_right_logical(ids16, threes)
        for rr in range(LANES):
            b = jnp.max(jnp.where(lanes == rr, b16, zeros))
            pltpu.async_copy(w_hbm.at[b], buf.at[rr], sem)

    def drain(buf, sem):
        pltpu.make_async_copy(w_hbm.at[pl.ds(0, LANES)], buf, sem).wait()

    def extract(c, buf):
        ids16 = ids_v[pl.ds(c * LANES, LANES)]
        w16 = jnp.bitwise_and(ids16, sevens)
        for rr in range(LANES):
            w = jnp.max(jnp.where(lanes == rr, w16, zeros))
            row = jnp.full((LANES,), rr, jnp.int32)
            wv = jnp.full((LANES,), w, jnp.int32)
            g = jnp.full((LANES,), 0, jnp.int32) + c * LANES + rr
            for k in range(DIM // LANES):
                cols = lanes + (k * LANES)
                vals = plsc.load_gather(buf, [row, wv, cols])
                plsc.store_scatter(out_v, [g, cols + 1], vals)

    for half in range(2):
        base = wid * BPW + half * HALF

        pltpu.sync_copy(x_hbm.at[pl.ds(base, HALF)], x_v)

        for i in range(NCH):
            r = lanes + (i * LANES)
            vals = plsc.load_gather(x_v, [r, col1])
            ids_v[pl.ds(i * LANES, LANES)] = vals.astype(jnp.int32)

        fire(0, blk0, sem0)

        def pass_body(i, carry):
            r = lanes + i * LANES
            for c in range(F):
                if c == 1:
                    continue
                dst = 0 if c == 0 else DIM - 1 + c
                vals = plsc.load_gather(
                    x_v, [r, jnp.full((LANES,), c, jnp.int32)])
                plsc.store_scatter(
                    out_v, [r, jnp.full((LANES,), dst, jnp.int32)], vals)
            return carry

        lax.fori_loop(0, NCH, pass_body, 0)

        def ring_body(i, carry):
            c0 = 2 * i
            fire(c0 + 1, blk1, sem1)
            drain(blk0, sem0)
            extract(c0, blk0)

            @pl.when(i < NCH // 2 - 1)
            def _():
                fire(c0 + 2, blk0, sem0)

            drain(blk1, sem1)
            extract(c0 + 1, blk1)
            return carry

        lax.fori_loop(0, NCH // 2, ring_body, 0)

        pltpu.sync_copy(out_v, out_hbm.at[pl.ds(base, HALF)])


@jax.jit
def kernel(x, W):
    mesh = plsc.VectorSubcoreMesh(core_axis_name="c", subcore_axis_name="s")
    fused = functools.partial(
        pl.kernel,
        mesh=mesh,
        compiler_params=pltpu.CompilerParams(
            needs_layout_passes=False, use_tc_tiling_on_sc=True),
        out_type=jax.ShapeDtypeStruct((B, OUT_F), jnp.float32),
        scratch_types=[
            pltpu.VMEM((HALF, F), jnp.float32),
            pltpu.VMEM((HALF,), jnp.int32),
            pltpu.VMEM((LANES, TB, DIM), jnp.float32),
            pltpu.VMEM((LANES, TB, DIM), jnp.float32),
            pltpu.VMEM((HALF, OUT_F), jnp.float32),
            pltpu.SemaphoreType.DMA,
            pltpu.SemaphoreType.DMA,
        ],
    )(_emb_body)
    return fused(x, W.reshape(NBLK, TB, DIM))
